# baseline (speedup 1.0000x reference)
import functools

import jax
import jax.numpy as jnp
from jax import lax
from jax.experimental import pallas as pl
from jax.experimental.pallas import tpu as pltpu
from jax.experimental.pallas import tpu_sc as plsc

VOCAB = 100000
EMBED = 300
NUM_IDX = 1024 * 50

_NC = 2
_NS = 16
_NW = _NC * _NS

PER_W = NUM_IDX // _NW
CHUNK = 80
NBUF = 4
ROUNDS = PER_W // (CHUNK * NBUF)

_mesh = plsc.VectorSubcoreMesh(core_axis_name="c", subcore_axis_name="s")


@functools.partial(
    pl.kernel,
    mesh=_mesh,
    out_type=jax.ShapeDtypeStruct((NUM_IDX, EMBED), jnp.float32),
    compiler_params=pltpu.CompilerParams(use_tc_tiling_on_sc=False),
    scratch_types=[
        pltpu.VMEM((PER_W,), jnp.int32),
        pltpu.VMEM((NBUF, CHUNK, EMBED), jnp.float32),
        pltpu.SemaphoreType.DMA,
        pltpu.SemaphoreType.DMA((NBUF,)),
        pltpu.SemaphoreType.DMA((NBUF,)),
    ],
)
def _gather_sc(table_hbm, idx_hbm, out_hbm, idx_v, rows_v, sem_idx, sem_in,
               sem_out):
    wid = lax.axis_index("s") * _NC + lax.axis_index("c")
    base = wid * PER_W

    pltpu.async_copy(idx_hbm.at[pl.ds(base, PER_W)], idx_v, sem_idx).wait()

    def wait_out(b, chunk):
        pltpu.make_async_copy(
            rows_v.at[b],
            out_hbm.at[pl.ds(base + chunk * CHUNK, CHUNK)],
            sem_out.at[b],
        ).wait()

    def round_body(g, carry):
        for b in range(NBUF):
            chunk = g * NBUF + b

            @pl.when(g > 0)
            def _():
                wait_out(b, chunk - NBUF)

            off = chunk * CHUNK
            for r16 in range(CHUNK // 16):
                ivec = idx_v[pl.ds(off + r16 * 16, 16)]
                for u in range(16):
                    pltpu.async_copy(
                        table_hbm.at[pl.ds(ivec[u], 1)],
                        rows_v.at[b].at[pl.ds(r16 * 16 + u, 1)],
                        sem_in.at[b],
                    )
            pltpu.make_async_copy(
                table_hbm.at[pl.ds(0, CHUNK)], rows_v.at[b], sem_in.at[b]
            ).wait()
            pltpu.async_copy(
                rows_v.at[b],
                out_hbm.at[pl.ds(base + off, CHUNK)],
                sem_out.at[b],
            )
        return carry

    lax.fori_loop(0, ROUNDS, round_body, 0)
    for b in range(NBUF):
        wait_out(b, (ROUNDS - 1) * NBUF + b)


def kernel(inp, table):
    idx = inp.reshape(-1)
    out = _gather_sc(table, idx)
    return out.reshape(inp.shape[0], inp.shape[1], EMBED)
